# baseline (speedup 1.0000x reference)
import jax
import jax.numpy as jnp
from jax.experimental import pallas as pl
from jax.experimental.pallas import tpu as pltpu


def _body(x_ref, o_ref):
    o_ref[:, :, :, 0:128] = x_ref[:, :, 0, :, :]
    o_ref[:, :, :, 128:256] = x_ref[:, :, 1, :, :]


def kernel(batch):
    B, H, W, C = batch.shape
    k = 2
    Ho, Wo = H // k, W // k
    Co = k * k * C
    x = batch.reshape(B, Ho, k, Wo, k * C)

    T = 16
    grid = (B, Ho // T)

    out = pl.pallas_call(
        _body,
        grid=grid,
        in_specs=[
            pl.BlockSpec((1, T, k, Wo, k * C), lambda b, i: (b, i, 0, 0, 0)),
        ],
        out_specs=pl.BlockSpec((1, T, Wo, Co), lambda b, i: (b, i, 0, 0)),
        out_shape=jax.ShapeDtypeStruct((B, Ho, Wo, Co), batch.dtype),
        compiler_params=pltpu.CompilerParams(
            dimension_semantics=("parallel", "arbitrary"),
        ),
    )(x)
    return out


# speedup vs baseline: 1.2598x; 1.2598x over previous
import jax
import jax.numpy as jnp
from jax.experimental import pallas as pl
from jax.experimental.pallas import tpu as pltpu

_T = 16


def _body(x_ref, o_ref):
    evens = x_ref[:, :, pl.ds(0, 112, 2), :]
    odds = x_ref[:, :, pl.ds(1, 112, 2), :]
    e = evens.reshape(_T, 2, 112, 64)
    o = odds.reshape(_T, 2, 112, 64)
    o_ref[0, :, :, 0:64] = e[:, 0]
    o_ref[0, :, :, 64:128] = o[:, 0]
    o_ref[0, :, :, 128:192] = e[:, 1]
    o_ref[0, :, :, 192:256] = o[:, 1]


def kernel(batch):
    B, H, W, C = batch.shape
    k = 2
    Ho, Wo = H // k, W // k
    Co = k * k * C

    grid = (B, Ho // _T)

    out = pl.pallas_call(
        _body,
        grid=grid,
        in_specs=[
            pl.BlockSpec((1, k * _T, W, C), lambda b, i: (b, i, 0, 0)),
        ],
        out_specs=pl.BlockSpec((1, _T, Wo, Co), lambda b, i: (b, i, 0, 0)),
        out_shape=jax.ShapeDtypeStruct((B, Ho, Wo, Co), batch.dtype),
        compiler_params=pltpu.CompilerParams(
            dimension_semantics=("parallel", "arbitrary"),
        ),
    )(batch)
    return out


# speedup vs baseline: 1.2806x; 1.0164x over previous
import jax
import jax.numpy as jnp
from jax.experimental import pallas as pl
from jax.experimental.pallas import tpu as pltpu

_T = 56


def _body(x_ref, o_ref):
    evens = x_ref[:, :, pl.ds(0, 112, 2), :]
    odds = x_ref[:, :, pl.ds(1, 112, 2), :]
    e = evens.reshape(_T, 2, 112, 64)
    o = odds.reshape(_T, 2, 112, 64)
    o_ref[0, :, :, 0:64] = e[:, 0]
    o_ref[0, :, :, 64:128] = o[:, 0]
    o_ref[0, :, :, 128:192] = e[:, 1]
    o_ref[0, :, :, 192:256] = o[:, 1]


def kernel(batch):
    B, H, W, C = batch.shape
    k = 2
    Ho, Wo = H // k, W // k
    Co = k * k * C

    grid = (B, Ho // _T)

    out = pl.pallas_call(
        _body,
        grid=grid,
        in_specs=[
            pl.BlockSpec((1, k * _T, W, C), lambda b, i: (b, i, 0, 0)),
        ],
        out_specs=pl.BlockSpec((1, _T, Wo, Co), lambda b, i: (b, i, 0, 0)),
        out_shape=jax.ShapeDtypeStruct((B, Ho, Wo, Co), batch.dtype),
        compiler_params=pltpu.CompilerParams(
            dimension_semantics=("parallel", "arbitrary"),
        ),
    )(batch)
    return out
